# baseline (speedup 1.0000x reference)
import functools

import jax
import jax.numpy as jnp
from jax import lax
from jax.experimental import pallas as pl
from jax.experimental.pallas import tpu as pltpu
from jax.experimental.pallas import tpu_sc as plsc

N = 2097152
H = N // 2
OUT_COLS = 21

NC, NS = 2, 16
NW = NC * NS
PER_W = H // NW
CHUNK = 512
NCHUNK = PER_W // CHUNK

BR = 32768
GRID = H // BR


def _tc_body(x_ref, o_ref):
    s = jnp.sum(x_ref[...])
    o_ref[...] = jnp.full((BR, OUT_COLS), 1, jnp.int32) + s.astype(jnp.int32)


def _tc_half(feature):
    x2d = feature.reshape(GRID * 2, BR // 1024, 1024)
    return pl.pallas_call(
        _tc_body,
        grid=(GRID,),
        in_specs=[pl.BlockSpec((1, 8, 1024), lambda i: (i, 0, 0))],
        out_specs=pl.BlockSpec((BR, OUT_COLS), lambda i: (i, 0)),
        out_shape=jax.ShapeDtypeStruct((H, OUT_COLS), jnp.int32),
    )(x2d)


def _sc_body(feat_hbm, out_hbm, obuf, sem):
    wid = lax.axis_index("s") * NC + lax.axis_index("c")
    base = wid * PER_W
    zeros = jnp.zeros((16,), jnp.int32)

    @pl.loop(0, CHUNK)
    def _zero(r):
        obuf[r, pl.ds(0, 16)] = zeros
        obuf[r, pl.ds(OUT_COLS - 16, 16)] = zeros

    @pl.loop(0, NCHUNK)
    def _chunk(c):
        pltpu.async_copy(obuf, out_hbm.at[pl.ds(base + c * CHUNK, CHUNK)],
                         sem)

    @pl.loop(0, NCHUNK)
    def _drain(c):
        pltpu.make_async_copy(obuf,
                              out_hbm.at[pl.ds(base + c * CHUNK, CHUNK)],
                              sem).wait()


@functools.partial(
    pl.kernel,
    out_type=jax.ShapeDtypeStruct((H, OUT_COLS), jnp.int32),
    mesh=plsc.VectorSubcoreMesh(core_axis_name="c", subcore_axis_name="s"),
    compiler_params=pltpu.CompilerParams(needs_layout_passes=False),
    scratch_types=[
        pltpu.VMEM((CHUNK, OUT_COLS), jnp.int32),
        pltpu.SemaphoreType.DMA,
    ],
)
def _sc_half(feat_hbm, out_hbm, obuf, sem):
    _sc_body(feat_hbm, out_hbm, obuf, sem)


def kernel(feature):
    b = _sc_half(feature)
    a = _tc_half(feature)
    return jnp.concatenate([a, b], axis=0).astype(jnp.int64)


# speedup vs baseline: 1.0915x; 1.0915x over previous
import functools

import jax
import jax.numpy as jnp
from jax import lax
from jax.experimental import pallas as pl
from jax.experimental.pallas import tpu as pltpu
from jax.experimental.pallas import tpu_sc as plsc

N = 2097152
N_BINS = 20
OUT_COLS = N_BINS + 1

NC = 2
NS = 16
NW = NC * NS
PER_W = N // NW
CHUNK = 256
GROUPS = CHUNK // 16
NCHUNK = PER_W // CHUNK


def _minmax_tc_kernel(x_ref, mn_ref, mx_ref):
    mn_ref[0] = jnp.min(x_ref[...])
    mx_ref[0] = jnp.max(x_ref[...])


def _minmax(feature):
    x2d = feature.reshape(2048, 1024)
    return pl.pallas_call(
        _minmax_tc_kernel,
        out_shape=[
            jax.ShapeDtypeStruct((1,), jnp.float32),
            jax.ShapeDtypeStruct((1,), jnp.float32),
        ],
        out_specs=[
            pl.BlockSpec(memory_space=pltpu.SMEM),
            pl.BlockSpec(memory_space=pltpu.SMEM),
        ],
    )(x2d)


def _sc_body(feat_hbm, mn_hbm, mx_hbm, out_hbm,
             xbufs, obufs, ibufs, mnv, mxv, sxs, sos):
    wid = lax.axis_index("s") * NC + lax.axis_index("c")
    base = wid * PER_W

    pltpu.sync_copy(mn_hbm, mnv)
    pltpu.sync_copy(mx_hbm, mxv)
    mn = mnv[...]
    mx = mxv[...]
    scale = 18.0 / (mx - mn)

    ones = jnp.full((16,), 1, jnp.int32)
    zeros = jnp.zeros((16,), jnp.int32)
    lane = lax.iota(jnp.int32, 16)

    def x_copy(c, b):
        return pltpu.make_async_copy(
            feat_hbm.at[pl.ds(base + c * CHUNK, CHUNK)], xbufs[b], sxs[b])

    def o_copy(c, b):
        return pltpu.make_async_copy(
            obufs[b], out_hbm.at[pl.ds(base + c * CHUNK, CHUNK)], sos[b])

    for b in range(2):
        @pl.loop(0, CHUNK)
        def _zero(r, _b=b):
            obufs[_b][r, pl.ds(0, 16)] = zeros
            obufs[_b][r, pl.ds(OUT_COLS - 16, 16)] = zeros

    for b in range(2):
        x_copy(b, b).start()

    @pl.loop(0, NCHUNK, step=2)
    def _chunk2(c0):
        for b in range(2):
            c = c0 + b

            @pl.when(c >= 2)
            def _reclaim():
                o_copy(c - 2, b).wait()

                @pl.loop(0, GROUPS)
                def _restore(g):
                    idx = ibufs[b][pl.ds(g * 16, 16)]
                    plsc.store_scatter(obufs[b], [lane + g * 16, idx],
                                       zeros)

            x_copy(c, b).wait()

            @pl.loop(0, GROUPS)
            def _group(g):
                x = xbufs[b][pl.ds(g * 16, 16)]
                t = (x - mn) * scale
                j = t.astype(jnp.int32)
                idx = jnp.minimum(j + 1, N_BINS - 1)
                plsc.store_scatter(obufs[b], [lane + g * 16, idx], ones)
                ibufs[b][pl.ds(g * 16, 16)] = idx

            @pl.when(c + 2 < NCHUNK)
            def _prefetch():
                x_copy(c + 2, b).start()

            o_copy(c, b).start()

    for b in range(2):
        o_copy(NCHUNK - 2 + b, b).wait()


@functools.partial(
    pl.kernel,
    out_type=jax.ShapeDtypeStruct((N, OUT_COLS), jnp.int32),
    mesh=plsc.VectorSubcoreMesh(core_axis_name="c", subcore_axis_name="s"),
    compiler_params=pltpu.CompilerParams(needs_layout_passes=False),
    scratch_types=[
        pltpu.VMEM((CHUNK,), jnp.float32),
        pltpu.VMEM((CHUNK,), jnp.float32),
        pltpu.VMEM((CHUNK, OUT_COLS), jnp.int32),
        pltpu.VMEM((CHUNK, OUT_COLS), jnp.int32),
        pltpu.VMEM((CHUNK,), jnp.int32),
        pltpu.VMEM((CHUNK,), jnp.int32),
        pltpu.VMEM((16,), jnp.float32),
        pltpu.VMEM((16,), jnp.float32),
        pltpu.SemaphoreType.DMA,
        pltpu.SemaphoreType.DMA,
        pltpu.SemaphoreType.DMA,
        pltpu.SemaphoreType.DMA,
    ],
)
def _sc_onehot(feat_hbm, mn_hbm, mx_hbm, out_hbm,
               xbuf0, xbuf1, obuf0, obuf1, ibuf0, ibuf1, mnv, mxv,
               sx0, sx1, so0, so1):
    _sc_body(feat_hbm, mn_hbm, mx_hbm, out_hbm,
             (xbuf0, xbuf1), (obuf0, obuf1), (ibuf0, ibuf1),
             mnv, mxv, (sx0, sx1), (so0, so1))


def kernel(feature):
    if feature.ndim == 2 and feature.shape[1] == 1:
        feature = jnp.squeeze(feature, axis=1)
    mn, mx = _minmax(feature)
    mn16 = jnp.broadcast_to(mn, (16,))
    mx16 = jnp.broadcast_to(mx, (16,))
    out = _sc_onehot(feature, mn16, mx16)
    return out.astype(jnp.int64)


# speedup vs baseline: 1.0935x; 1.0018x over previous
import functools

import jax
import jax.numpy as jnp
from jax import lax
from jax.experimental import pallas as pl
from jax.experimental.pallas import tpu as pltpu
from jax.experimental.pallas import tpu_sc as plsc

N = 2097152
N_BINS = 20
OUT_COLS = N_BINS + 1

NC = 2
NS = 16
NW = NC * NS
PER_W = N // NW
CHUNK = 256
GROUPS = CHUNK // 16
NCHUNK = PER_W // CHUNK


def _minmax_tc_kernel(x_ref, mn_ref, mx_ref):
    mn_ref[...] = jnp.full((16,), jnp.min(x_ref[...]), jnp.float32)
    mx_ref[...] = jnp.full((16,), jnp.max(x_ref[...]), jnp.float32)


def _minmax(feature):
    x2d = feature.reshape(2048, 1024)
    return pl.pallas_call(
        _minmax_tc_kernel,
        out_shape=[
            jax.ShapeDtypeStruct((16,), jnp.float32),
            jax.ShapeDtypeStruct((16,), jnp.float32),
        ],
    )(x2d)


def _sc_body(feat_hbm, mn_hbm, mx_hbm, out_hbm,
             xbufs, obufs, ibufs, mnv, mxv, sxs, sos):
    wid = lax.axis_index("s") * NC + lax.axis_index("c")
    base = wid * PER_W

    pltpu.sync_copy(mn_hbm, mnv)
    pltpu.sync_copy(mx_hbm, mxv)
    mn = mnv[...]
    mx = mxv[...]
    scale = 18.0 / (mx - mn)

    ones = jnp.full((16,), 1, jnp.int32)
    zeros = jnp.zeros((16,), jnp.int32)
    lane = lax.iota(jnp.int32, 16)

    def x_copy(c, b):
        return pltpu.make_async_copy(
            feat_hbm.at[pl.ds(base + c * CHUNK, CHUNK)], xbufs[b], sxs[b])

    def o_copy(c, b):
        return pltpu.make_async_copy(
            obufs[b], out_hbm.at[pl.ds(base + c * CHUNK, CHUNK)], sos[b])

    for b in range(2):
        @pl.loop(0, CHUNK)
        def _zero(r, _b=b):
            obufs[_b][r, pl.ds(0, 16)] = zeros
            obufs[_b][r, pl.ds(OUT_COLS - 16, 16)] = zeros

    for b in range(2):
        x_copy(b, b).start()

    @pl.loop(0, NCHUNK, step=2)
    def _chunk2(c0):
        for b in range(2):
            c = c0 + b

            @pl.when(c >= 2)
            def _reclaim():
                o_copy(c - 2, b).wait()

                @pl.loop(0, GROUPS)
                def _restore(g):
                    idx = ibufs[b][pl.ds(g * 16, 16)]
                    plsc.store_scatter(obufs[b], [lane + g * 16, idx],
                                       zeros)

            x_copy(c, b).wait()

            @pl.loop(0, GROUPS)
            def _group(g):
                x = xbufs[b][pl.ds(g * 16, 16)]
                t = (x - mn) * scale
                j = t.astype(jnp.int32)
                idx = jnp.minimum(j + 1, N_BINS - 1)
                plsc.store_scatter(obufs[b], [lane + g * 16, idx], ones)
                ibufs[b][pl.ds(g * 16, 16)] = idx

            @pl.when(c + 2 < NCHUNK)
            def _prefetch():
                x_copy(c + 2, b).start()

            o_copy(c, b).start()

    for b in range(2):
        o_copy(NCHUNK - 2 + b, b).wait()


@functools.partial(
    pl.kernel,
    out_type=jax.ShapeDtypeStruct((N, OUT_COLS), jnp.int32),
    mesh=plsc.VectorSubcoreMesh(core_axis_name="c", subcore_axis_name="s"),
    compiler_params=pltpu.CompilerParams(needs_layout_passes=False),
    scratch_types=[
        pltpu.VMEM((CHUNK,), jnp.float32),
        pltpu.VMEM((CHUNK,), jnp.float32),
        pltpu.VMEM((CHUNK, OUT_COLS), jnp.int32),
        pltpu.VMEM((CHUNK, OUT_COLS), jnp.int32),
        pltpu.VMEM((CHUNK,), jnp.int32),
        pltpu.VMEM((CHUNK,), jnp.int32),
        pltpu.VMEM((16,), jnp.float32),
        pltpu.VMEM((16,), jnp.float32),
        pltpu.SemaphoreType.DMA,
        pltpu.SemaphoreType.DMA,
        pltpu.SemaphoreType.DMA,
        pltpu.SemaphoreType.DMA,
    ],
)
def _sc_onehot(feat_hbm, mn_hbm, mx_hbm, out_hbm,
               xbuf0, xbuf1, obuf0, obuf1, ibuf0, ibuf1, mnv, mxv,
               sx0, sx1, so0, so1):
    _sc_body(feat_hbm, mn_hbm, mx_hbm, out_hbm,
             (xbuf0, xbuf1), (obuf0, obuf1), (ibuf0, ibuf1),
             mnv, mxv, (sx0, sx1), (so0, so1))


def kernel(feature):
    if feature.ndim == 2 and feature.shape[1] == 1:
        feature = jnp.squeeze(feature, axis=1)
    mn16, mx16 = _minmax(feature)
    out = _sc_onehot(feature, mn16, mx16)
    return out.astype(jnp.int64)
